# baseline (speedup 1.0000x reference)
import jax
import jax.numpy as jnp
from jax import lax
from jax.experimental import pallas as pl
from jax.experimental.pallas import tpu as pltpu
from jax.experimental.pallas import tpu_sc as plsc

NB_STATES = 1000000
BATCH = 16384
LANES = 16
NUM_WORKERS = 32
B_PER_W = BATCH // NUM_WORKERS
NCHUNK = 4
CH = B_PER_W // NCHUNK

LN2 = 0.6931471805599453


def _log_f32(v):
    bits = lax.bitcast_convert_type(v, jnp.int32)
    e = (bits >> 23) - 127
    m = lax.bitcast_convert_type(
        (bits & 0x007FFFFF) | 0x3F800000, jnp.float32)
    z = m - 1.0
    s = z / (2.0 + z)
    s2 = s * s
    p = 2.0 * s * (1.0 + s2 * (1.0 / 3.0 + s2 * (0.2 + s2 * (1.0 / 7.0
                                                             + s2 / 9.0))))
    return e.astype(jnp.float32) * LN2 + p


def _sc_body(x_hbm, a_hbm, rho_hbm, theta_hbm, out_hbm,
             idx_v, a_v, tm_v, r_v, out_v, sem_a, sem_out, sems_i, sems_g):
    wid = lax.axis_index("s") * 2 + lax.axis_index("c")
    base = wid * B_PER_W
    ca = pltpu.async_copy(a_hbm.at[pl.ds(base, B_PER_W)], a_v, sem_a)
    idx_copies = [
        pltpu.async_copy(x_hbm.at[pl.ds(base + c * CH, CH)],
                         idx_v.at[pl.ds(c * CH, CH)], sems_i.at[c])
        for c in range(NCHUNK)
    ]
    gathers = []
    for c in range(NCHUNK):
        idx_copies[c].wait()
        sl = pl.ds(c * CH, CH)
        gt = pltpu.async_copy(theta_hbm.at[idx_v.at[sl]], tm_v.at[sl],
                              sems_g.at[c])
        gr = pltpu.async_copy(rho_hbm.at[idx_v.at[sl]], r_v.at[sl],
                              sems_g.at[c])
        gathers.append((gt, gr))
    ca.wait()
    out_copies = []
    for c in range(NCHUNK):
        gt, gr = gathers[c]
        gt.wait()
        gr.wait()
        for i in range(CH // LANES):
            sl = pl.ds(c * CH + i * LANES, LANES)
            t = tm_v[sl]
            r = r_v[sl]
            av = a_v[sl]
            congestion = 2.0 * _log_f32(r + 1e-05)
            move = jnp.where(av != 0, jnp.float32(0.1), jnp.float32(0.0))
            out_v[sl] = 100.0 * t - congestion - move
        out_copies.append(
            pltpu.async_copy(out_v.at[pl.ds(c * CH, CH)],
                             out_hbm.at[pl.ds(base + c * CH, CH)], sem_out))
    for c in range(NCHUNK):
        out_copies[c].wait()


@jax.jit
def kernel(x, a, rho, theta_map):
    mesh = plsc.VectorSubcoreMesh(core_axis_name="c", subcore_axis_name="s")
    run = pl.kernel(
        _sc_body,
        mesh=mesh,
        out_type=jax.ShapeDtypeStruct((BATCH,), jnp.float32),
        scratch_types=[
            pltpu.VMEM((B_PER_W,), jnp.int32),
            pltpu.VMEM((B_PER_W,), jnp.int32),
            pltpu.VMEM((B_PER_W,), jnp.float32),
            pltpu.VMEM((B_PER_W,), jnp.float32),
            pltpu.VMEM((B_PER_W,), jnp.float32),
            pltpu.SemaphoreType.DMA,
            pltpu.SemaphoreType.DMA,
            pltpu.SemaphoreType.DMA((NCHUNK,)),
            pltpu.SemaphoreType.DMA((NCHUNK,)),
        ],
    )
    return run(x, a, rho, theta_map)


# speedup vs baseline: 1.0153x; 1.0153x over previous
import jax
import jax.numpy as jnp
from jax import lax
from jax.experimental import pallas as pl
from jax.experimental.pallas import tpu as pltpu
from jax.experimental.pallas import tpu_sc as plsc

NB_STATES = 1000000
BATCH = 16384
LANES = 16
NUM_WORKERS = 32
B_PER_W = BATCH // NUM_WORKERS
HALF = B_PER_W // 2

LN2 = 0.6931471805599453
C0 = 0.99949435
C1 = -0.49190028
C2 = 0.28945382
C3 = -0.13604193
C4 = 0.03215113


def _log_f32(v):
    bits = lax.bitcast_convert_type(v, jnp.int32)
    e = (bits >> 23) - 127
    m = lax.bitcast_convert_type(
        (bits & 0x007FFFFF) | 0x3F800000, jnp.float32)
    z = m - 1.0
    p = z * (C0 + z * (C1 + z * (C2 + z * (C3 + z * C4))))
    return e.astype(jnp.float32) * LN2 + p


def _sc_body(x_hbm, a_hbm, rho_hbm, theta_hbm, out_hbm,
             idx_v, a_v, tm_v, r_v, out_v, sem_a, sem_out, sems_g):
    wid = lax.axis_index("s") * 2 + lax.axis_index("c")
    base = wid * B_PER_W
    ca = pltpu.async_copy(a_hbm.at[pl.ds(base, B_PER_W)], a_v, sem_a)
    pltpu.sync_copy(x_hbm.at[pl.ds(base, B_PER_W)], idx_v)
    gathers = []
    for h in range(2):
        sl = pl.ds(h * HALF, HALF)
        gt = pltpu.async_copy(theta_hbm.at[idx_v.at[sl]], tm_v.at[sl],
                              sems_g.at[h])
        gr = pltpu.async_copy(rho_hbm.at[idx_v.at[sl]], r_v.at[sl],
                              sems_g.at[h])
        gathers.append((gt, gr))
    ca.wait()
    out_copies = []
    for h in range(2):
        gt, gr = gathers[h]
        gt.wait()
        gr.wait()
        for i in range(HALF // LANES):
            sl = pl.ds(h * HALF + i * LANES, LANES)
            t = tm_v[sl]
            r = r_v[sl]
            av = a_v[sl]
            congestion = 2.0 * _log_f32(r + 1e-05)
            move = jnp.where(av != 0, jnp.float32(0.1), jnp.float32(0.0))
            out_v[sl] = 100.0 * t - congestion - move
        out_copies.append(
            pltpu.async_copy(out_v.at[pl.ds(h * HALF, HALF)],
                             out_hbm.at[pl.ds(base + h * HALF, HALF)],
                             sem_out))
    for c in out_copies:
        c.wait()


@jax.jit
def kernel(x, a, rho, theta_map):
    mesh = plsc.VectorSubcoreMesh(core_axis_name="c", subcore_axis_name="s")
    run = pl.kernel(
        _sc_body,
        mesh=mesh,
        out_type=jax.ShapeDtypeStruct((BATCH,), jnp.float32),
        scratch_types=[
            pltpu.VMEM((B_PER_W,), jnp.int32),
            pltpu.VMEM((B_PER_W,), jnp.int32),
            pltpu.VMEM((B_PER_W,), jnp.float32),
            pltpu.VMEM((B_PER_W,), jnp.float32),
            pltpu.VMEM((B_PER_W,), jnp.float32),
            pltpu.SemaphoreType.DMA,
            pltpu.SemaphoreType.DMA,
            pltpu.SemaphoreType.DMA((2,)),
        ],
    )
    return run(x, a, rho, theta_map)
